# Initial kernel scaffold; baseline (speedup 1.0000x reference)
import jax
import jax.numpy as jnp
from jax import lax
from jax.experimental import pallas as pl
from jax.experimental.pallas import tpu as pltpu
from jax.experimental.pallas import tpu_sc as plsc

N = 10000
E = 320000
H = 128
DIN = 196
EF = 7
NLAYERS = 3
HC = 144
NC, NS = 2, 16
NW = NC * NS
EW = E // NW
K = 80
ITERS = EW // K

_mesh = plsc.VectorSubcoreMesh(
    core_axis_name="c", subcore_axis_name="s", num_cores=NC, num_subcores=NS)


def _gather_body(hc_hbm, src_hbm, dst_hbm, ged_hbm, ges_hbm,
                 idx_s, idx_d, rows_s, rows_d, sem):
    c = lax.axis_index("c")
    s = lax.axis_index("s")
    wid = s * NC + c
    base_w = wid * EW

    def step(i, carry):
        base = base_w + i * K
        pltpu.sync_copy(dst_hbm.at[pl.ds(base, K)], idx_d)
        pltpu.sync_copy(src_hbm.at[pl.ds(base, K)], idx_s)
        pltpu.async_copy(hc_hbm.at[idx_d], rows_d, sem).wait()
        pltpu.async_copy(hc_hbm.at[idx_s], rows_s, sem).wait()
        pltpu.sync_copy(rows_d, ged_hbm.at[pl.ds(base, K)])
        pltpu.sync_copy(rows_s, ges_hbm.at[pl.ds(base, K)])
        return carry

    lax.fori_loop(0, ITERS, step, 0)


_gather = pl.kernel(
    _gather_body,
    out_type=(jax.ShapeDtypeStruct((E, HC), jnp.float32),
              jax.ShapeDtypeStruct((E, HC), jnp.float32)),
    mesh=_mesh,
    scratch_types=[
        pltpu.VMEM((K,), jnp.int32),
        pltpu.VMEM((K,), jnp.int32),
        pltpu.VMEM((K, HC), jnp.float32),
        pltpu.VMEM((K, HC), jnp.float32),
        pltpu.SemaphoreType.DMA,
    ],
)


def _scatter_body(mw_hbm, dst_hbm, zero_hbm, part_hbm,
                  idx_v, rows_v, agg_shared, sem):
    c = lax.axis_index("c")
    s = lax.axis_index("s")
    wid = s * NC + c
    rpt = N // NS

    pltpu.sync_copy(zero_hbm.at[pl.ds(s * rpt, rpt)],
                    agg_shared.at[pl.ds(s * rpt, rpt)])
    plsc.subcore_barrier()

    def step(i, carry):
        base = wid * EW + i * K
        pltpu.sync_copy(dst_hbm.at[pl.ds(base, K)], idx_v)
        pltpu.sync_copy(mw_hbm.at[pl.ds(base, K)], rows_v)
        pltpu.sync_copy(rows_v, agg_shared.at[idx_v], add=True)
        return carry

    lax.fori_loop(0, ITERS, step, 0)
    plsc.subcore_barrier()
    pltpu.sync_copy(agg_shared.at[pl.ds(s * rpt, rpt)],
                    part_hbm.at[pl.ds(c * N + s * rpt, rpt)])


_scatter = pl.kernel(
    _scatter_body,
    out_type=jax.ShapeDtypeStruct((NC * N, HC), jnp.float32),
    mesh=_mesh,
    scratch_types=[
        pltpu.VMEM((K,), jnp.int32),
        pltpu.VMEM((K, HC), jnp.float32),
        pltpu.VMEM_SHARED((N, HC), jnp.float32),
        pltpu.SemaphoreType.DMA,
    ],
)


def _proj_body(x_ref, cp_ref, w_ref, b_ref, out_ref):
    h = jnp.dot(x_ref[...], w_ref[...], preferred_element_type=jnp.float32)
    h = h + b_ref[...]
    out_ref[...] = jnp.concatenate([h, cp_ref[...]], axis=1)


def _proj(x, cp, w, b):
    r = 400
    return pl.pallas_call(
        _proj_body,
        grid=(N // r,),
        in_specs=[
            pl.BlockSpec((r, DIN), lambda i: (i, 0)),
            pl.BlockSpec((r, 16), lambda i: (i, 0)),
            pl.BlockSpec((DIN, H), lambda i: (0, 0)),
            pl.BlockSpec((1, H), lambda i: (0, 0)),
        ],
        out_specs=pl.BlockSpec((r, HC), lambda i: (i, 0)),
        out_shape=jax.ShapeDtypeStruct((N, HC), jnp.float32),
    )(x, cp, w, b)


def _edge_body(ged_ref, ges_ref, ea_ref, whd_ref, whs_ref, wd2_ref, wea_ref,
               b1_ref, w2_ref, b2_ref, xw1_ref, xb1_ref, xw2r_ref, out_ref):
    hd = ged_ref[:, :H]
    cd = ged_ref[:, H:]
    hs = ges_ref[:, :H]
    cs = ges_ref[:, H:]
    rel = cd - cs
    d2 = jnp.sum(rel * rel, axis=1, keepdims=True)
    t = (jnp.dot(hd, whd_ref[...], preferred_element_type=jnp.float32)
         + jnp.dot(hs, whs_ref[...], preferred_element_type=jnp.float32)
         + d2 * wd2_ref[...]
         + jnp.dot(ea_ref[...], wea_ref[...], preferred_element_type=jnp.float32)
         + b1_ref[...])
    m = jax.nn.silu(t)
    m = jax.nn.silu(
        jnp.dot(m, w2_ref[...], preferred_element_type=jnp.float32) + b2_ref[...])
    u = jax.nn.silu(
        jnp.dot(m, xw1_ref[...], preferred_element_type=jnp.float32) + xb1_ref[...])
    cw = jnp.sum(u * xw2r_ref[...], axis=1, keepdims=True)
    wrel = rel * (cw / (jnp.sqrt(d2) + 1.0))
    lane = lax.broadcasted_iota(jnp.int32, wrel.shape, 1)
    wrel = jnp.where(lane == 3, 1.0, wrel)
    out_ref[...] = jnp.concatenate([m, wrel], axis=1)


def _edge(ged, ges, ea, whd, whs, wd2, wea, b1, w2, b2, xw1, xb1, xw2r):
    be = 1280
    full = lambda shape: pl.BlockSpec(shape, lambda i: (0, 0))
    return pl.pallas_call(
        _edge_body,
        grid=(E // be,),
        in_specs=[
            pl.BlockSpec((be, HC), lambda i: (i, 0)),
            pl.BlockSpec((be, HC), lambda i: (i, 0)),
            pl.BlockSpec((be, 8), lambda i: (i, 0)),
            full((H, H)), full((H, H)), full((1, H)), full((8, H)),
            full((1, H)), full((H, H)), full((1, H)),
            full((H, H)), full((1, H)), full((1, H)),
        ],
        out_specs=pl.BlockSpec((be, HC), lambda i: (i, 0)),
        out_shape=jax.ShapeDtypeStruct((E, HC), jnp.float32),
    )(ged, ges, ea, whd, whs, wd2, wea, b1, w2, b2, xw1, xb1, xw2r)


def _node_body(hc_ref, p0_ref, p1_ref, w1a_ref, w1b_ref, b1_ref,
               w2_ref, b2_ref, out_ref):
    h = hc_ref[:, :H]
    cp = hc_ref[:, H:]
    sagg = p0_ref[...] + p1_ref[...]
    aggm = sagg[:, :H]
    wx = sagg[:, H:]
    deg = wx[:, 3:4]
    lane = lax.broadcasted_iota(jnp.int32, wx.shape, 1)
    aggx = jnp.where(lane < 3, wx, 0.0)
    cp_new = cp + aggx / (deg + 1.0)
    hu = jax.nn.silu(
        jnp.dot(h, w1a_ref[...], preferred_element_type=jnp.float32)
        + jnp.dot(aggm, w1b_ref[...], preferred_element_type=jnp.float32)
        + b1_ref[...])
    h_new = h + jnp.dot(hu, w2_ref[...], preferred_element_type=jnp.float32) \
        + b2_ref[...]
    out_ref[...] = jnp.concatenate([h_new, cp_new], axis=1)


def _node(hc, p0, p1, w1a, w1b, b1, w2, b2):
    r = 400
    full = lambda shape: pl.BlockSpec(shape, lambda i: (0, 0))
    return pl.pallas_call(
        _node_body,
        grid=(N // r,),
        in_specs=[
            pl.BlockSpec((r, HC), lambda i: (i, 0)),
            pl.BlockSpec((r, HC), lambda i: (i, 0)),
            pl.BlockSpec((r, HC), lambda i: (i, 0)),
            full((H, H)), full((H, H)), full((1, H)),
            full((H, H)), full((1, H)),
        ],
        out_specs=pl.BlockSpec((r, HC), lambda i: (i, 0)),
        out_shape=jax.ShapeDtypeStruct((N, HC), jnp.float32),
    )(hc, p0, p1, w1a, w1b, b1, w2, b2)


@jax.jit
def kernel(x, coords, edge_index, edge_attr, proj_w, proj_b,
           ew1, eb1, ew2, eb2, xw1, xb1, xw2, hw1, hb1, hw2, hb2):
    src = edge_index[0].astype(jnp.int32)
    dst = edge_index[1].astype(jnp.int32)
    cp0 = jnp.pad(coords, ((0, 0), (0, 16 - 3)))
    eap = jnp.pad(edge_attr, ((0, 0), (0, 8 - EF)))
    zero = jnp.zeros((N, HC), jnp.float32)

    hc = _proj(x, cp0, proj_w, proj_b.reshape(1, H))
    for i in range(NLAYERS):
        whd = ew1[i, :H]
        whs = ew1[i, H:2 * H]
        wd2 = ew1[i, 2 * H:2 * H + 1]
        wea = jnp.pad(ew1[i, 2 * H + 1:], ((0, 1), (0, 0)))
        ged, ges = _gather(hc, src, dst)
        mw = _edge(ged, ges, eap, whd, whs, wd2, wea,
                   eb1[i].reshape(1, H), ew2[i], eb2[i].reshape(1, H),
                   xw1[i], xb1[i].reshape(1, H), xw2[i].reshape(1, H))
        part = _scatter(mw, dst, zero)
        hc = _node(hc, part[:N], part[N:],
                   hw1[i, :H], hw1[i, H:], hb1[i].reshape(1, H),
                   hw2[i], hb2[i].reshape(1, H))
    return hc[:, H:H + 3]


# speedup vs baseline: 2.8042x; 2.8042x over previous
import jax
import jax.numpy as jnp
from jax import lax
from jax.experimental import pallas as pl
from jax.experimental.pallas import tpu as pltpu
from jax.experimental.pallas import tpu_sc as plsc

N = 10000
E = 320000
H = 128
DIN = 196
EF = 7
NLAYERS = 3
NC, NS = 2, 16
NW = NC * NS
K = 80
NCHUNK = E // K
JITERS = NCHUNK // NW
BE = 512
NBLK = E // BE

_mesh = plsc.VectorSubcoreMesh(
    core_axis_name="c", subcore_axis_name="s", num_cores=NC, num_subcores=NS)
_sc_params = pltpu.CompilerParams(needs_layout_passes=False)


def _gather_body(h_hbm, cx_hbm, cy_hbm, cz_hbm, src_hbm, dst_hbm,
                 ged_hbm, ges_hbm, rx_hbm, ry_hbm, rz_hbm, d2_hbm,
                 cxv, cyv, czv, idxs, idxd, rowsd, rowss,
                 rxv, ryv, rzv, d2v, sem):
    wid = lax.axis_index("s") * NC + lax.axis_index("c")
    pltpu.sync_copy(cx_hbm, cxv)
    pltpu.sync_copy(cy_hbm, cyv)
    pltpu.sync_copy(cz_hbm, czv)

    def step(j, carry):
        base = (wid + NW * j) * K
        a = pltpu.async_copy(dst_hbm.at[pl.ds(base, K)], idxd, sem)
        b = pltpu.async_copy(src_hbm.at[pl.ds(base, K)], idxs, sem)
        a.wait()
        b.wait()
        g1 = pltpu.async_copy(h_hbm.at[idxd], rowsd, sem)
        g2 = pltpu.async_copy(h_hbm.at[idxs], rowss, sem)
        for jj in range(K // 16):
            sl = pl.ds(jj * 16, 16)
            vd = idxd[sl]
            vs = idxs[sl]
            rx = plsc.load_gather(cxv, [vd]) - plsc.load_gather(cxv, [vs])
            ry = plsc.load_gather(cyv, [vd]) - plsc.load_gather(cyv, [vs])
            rz = plsc.load_gather(czv, [vd]) - plsc.load_gather(czv, [vs])
            rxv[sl] = rx
            ryv[sl] = ry
            rzv[sl] = rz
            d2v[sl] = rx * rx + ry * ry + rz * rz
        g1.wait()
        g2.wait()
        w1 = pltpu.async_copy(rowsd, ged_hbm.at[pl.ds(base, K)], sem)
        w2 = pltpu.async_copy(rowss, ges_hbm.at[pl.ds(base, K)], sem)
        w3 = pltpu.async_copy(rxv, rx_hbm.at[pl.ds(base, K)], sem)
        w4 = pltpu.async_copy(ryv, ry_hbm.at[pl.ds(base, K)], sem)
        w5 = pltpu.async_copy(rzv, rz_hbm.at[pl.ds(base, K)], sem)
        w6 = pltpu.async_copy(d2v, d2_hbm.at[pl.ds(base, K)], sem)
        w1.wait()
        w2.wait()
        w3.wait()
        w4.wait()
        w5.wait()
        w6.wait()
        return carry

    lax.fori_loop(0, JITERS, step, 0)


_gather = pl.kernel(
    _gather_body,
    out_type=(jax.ShapeDtypeStruct((E, H), jnp.float32),
              jax.ShapeDtypeStruct((E, H), jnp.float32),
              jax.ShapeDtypeStruct((E,), jnp.float32),
              jax.ShapeDtypeStruct((E,), jnp.float32),
              jax.ShapeDtypeStruct((E,), jnp.float32),
              jax.ShapeDtypeStruct((E,), jnp.float32)),
    mesh=_mesh,
    scratch_types=[
        pltpu.VMEM((N,), jnp.float32),
        pltpu.VMEM((N,), jnp.float32),
        pltpu.VMEM((N,), jnp.float32),
        pltpu.VMEM((K,), jnp.int32),
        pltpu.VMEM((K,), jnp.int32),
        pltpu.VMEM((K, H), jnp.float32),
        pltpu.VMEM((K, H), jnp.float32),
        pltpu.VMEM((K,), jnp.float32),
        pltpu.VMEM((K,), jnp.float32),
        pltpu.VMEM((K,), jnp.float32),
        pltpu.VMEM((K,), jnp.float32),
        pltpu.SemaphoreType.DMA,
    ],
    compiler_params=_sc_params,
)


def _scatter_body(m_hbm, wx_hbm, wy_hbm, wz_hbm, dst_hbm, zm_hbm, z4_hbm,
                  partm_hbm, part2_hbm,
                  idxv, idx1, idx2, idx3, rowsv, wxv, wyv, wzv, onesv, buf4,
                  aggm_sh, agg4_sh, sem):
    c = lax.axis_index("c")
    s = lax.axis_index("s")
    wid = s * NC + c
    rpt = 624

    pltpu.sync_copy(zm_hbm.at[pl.ds(s * rpt, rpt)],
                    aggm_sh.at[pl.ds(s * rpt, rpt)])

    @pl.when(s == NS - 1)
    def _():
        pltpu.sync_copy(zm_hbm.at[pl.ds(NS * rpt, N - NS * rpt)],
                        aggm_sh.at[pl.ds(NS * rpt, N - NS * rpt)])

    @pl.when(s < 8)
    def _():
        pltpu.sync_copy(z4_hbm.at[pl.ds(s * 5000, 5000)], buf4)
        pltpu.sync_copy(buf4, agg4_sh.at[pl.ds(s * 5000, 5000)])

    for jj in range(K // 16):
        onesv[pl.ds(jj * 16, 16)] = jnp.full((16,), 1.0, jnp.float32)
    plsc.subcore_barrier()

    def step(j, carry):
        base = (wid + NW * j) * K
        a = pltpu.async_copy(dst_hbm.at[pl.ds(base, K)], idxv, sem)
        a.wait()
        b1 = pltpu.async_copy(m_hbm.at[pl.ds(base, K)], rowsv, sem)
        b2 = pltpu.async_copy(wx_hbm.at[pl.ds(base, K)], wxv, sem)
        b3 = pltpu.async_copy(wy_hbm.at[pl.ds(base, K)], wyv, sem)
        b4 = pltpu.async_copy(wz_hbm.at[pl.ds(base, K)], wzv, sem)
        for jj in range(K // 16):
            sl = pl.ds(jj * 16, 16)
            v = idxv[sl]
            idx1[sl] = v + N
            idx2[sl] = v + 2 * N
            idx3[sl] = v + 3 * N
        b1.wait()
        b2.wait()
        b3.wait()
        b4.wait()
        pltpu.sync_copy(rowsv, aggm_sh.at[idxv], add=True)
        pltpu.sync_copy(wxv, agg4_sh.at[idxv], add=True)
        pltpu.sync_copy(wyv, agg4_sh.at[idx1], add=True)
        pltpu.sync_copy(wzv, agg4_sh.at[idx2], add=True)
        pltpu.sync_copy(onesv, agg4_sh.at[idx3], add=True)
        return carry

    lax.fori_loop(0, JITERS, step, 0)
    plsc.subcore_barrier()
    pltpu.sync_copy(aggm_sh.at[pl.ds(s * rpt, rpt)],
                    partm_hbm.at[pl.ds(c * N + s * rpt, rpt)])

    @pl.when(s == NS - 1)
    def _():
        pltpu.sync_copy(aggm_sh.at[pl.ds(NS * rpt, N - NS * rpt)],
                        partm_hbm.at[pl.ds(c * N + NS * rpt, N - NS * rpt)])

    @pl.when(s < 8)
    def _():
        pltpu.sync_copy(agg4_sh.at[pl.ds(s * 5000, 5000)], buf4)
        pltpu.sync_copy(buf4, part2_hbm.at[pl.ds(c * 4 * N + s * 5000, 5000)])


_scatter = pl.kernel(
    _scatter_body,
    out_type=(jax.ShapeDtypeStruct((NC * N, H), jnp.float32),
              jax.ShapeDtypeStruct((NC * 4 * N,), jnp.float32)),
    mesh=_mesh,
    scratch_types=[
        pltpu.VMEM((K,), jnp.int32),
        pltpu.VMEM((K,), jnp.int32),
        pltpu.VMEM((K,), jnp.int32),
        pltpu.VMEM((K,), jnp.int32),
        pltpu.VMEM((K, H), jnp.float32),
        pltpu.VMEM((K,), jnp.float32),
        pltpu.VMEM((K,), jnp.float32),
        pltpu.VMEM((K,), jnp.float32),
        pltpu.VMEM((K,), jnp.float32),
        pltpu.VMEM((5000,), jnp.float32),
        pltpu.VMEM_SHARED((N, H), jnp.float32),
        pltpu.VMEM_SHARED((4 * N,), jnp.float32),
        pltpu.SemaphoreType.DMA,
    ],
    compiler_params=_sc_params,
)


def _proj_body(x_ref, w_ref, b_ref, out_ref):
    out_ref[...] = jnp.dot(x_ref[...], w_ref[...],
                           preferred_element_type=jnp.float32) + b_ref[...]


def _proj(x, w, b):
    r = 400
    return pl.pallas_call(
        _proj_body,
        grid=(N // r,),
        in_specs=[
            pl.BlockSpec((r, DIN), lambda i: (i, 0)),
            pl.BlockSpec((DIN, H), lambda i: (0, 0)),
            pl.BlockSpec((1, H), lambda i: (0, 0)),
        ],
        out_specs=pl.BlockSpec((r, H), lambda i: (i, 0)),
        out_shape=jax.ShapeDtypeStruct((N, H), jnp.float32),
    )(x, w, b)


def _edge_body(ged_ref, ges_ref, rx_ref, ry_ref, rz_ref, d2_ref, eat_ref,
               whd_ref, whs_ref, wd2_ref,
               wea_ref, b1_ref, w2_ref, b2_ref, xw1_ref, xb1_ref, xw2r_ref,
               m_ref, wx_ref, wy_ref, wz_ref):
    rd4 = jnp.concatenate(
        [rx_ref[...][None, :], ry_ref[...][None, :],
         rz_ref[...][None, :], d2_ref[...][None, :]], axis=0)
    rdt = jnp.transpose(rd4)
    eat = jnp.transpose(eat_ref[0])
    d2 = rdt[:, 3:4]
    t = (jnp.dot(ged_ref[...], whd_ref[...], preferred_element_type=jnp.float32)
         + jnp.dot(ges_ref[...], whs_ref[...], preferred_element_type=jnp.float32)
         + d2 * wd2_ref[...]
         + jnp.dot(eat, wea_ref[...], preferred_element_type=jnp.float32)
         + b1_ref[...])
    m = jax.nn.silu(t)
    m = jax.nn.silu(
        jnp.dot(m, w2_ref[...], preferred_element_type=jnp.float32) + b2_ref[...])
    u = jax.nn.silu(
        jnp.dot(m, xw1_ref[...], preferred_element_type=jnp.float32) + xb1_ref[...])
    cw = jnp.sum(u * xw2r_ref[...], axis=1, keepdims=True)
    wr4 = jnp.transpose(rdt * (cw / (jnp.sqrt(d2) + 1.0)))
    m_ref[...] = m
    wx_ref[...] = wr4[0]
    wy_ref[...] = wr4[1]
    wz_ref[...] = wr4[2]


def _edge(ged, ges, rx, ry, rz, d2, eat,
          whd, whs, wd2, wea, b1, w2, b2, xw1, xb1, xw2r):
    full = lambda shape: pl.BlockSpec(shape, lambda i: (0,) * len(shape))
    vec = pl.BlockSpec((BE,), lambda i: (i,))
    return pl.pallas_call(
        _edge_body,
        grid=(NBLK,),
        in_specs=[
            pl.BlockSpec((BE, H), lambda i: (i, 0)),
            pl.BlockSpec((BE, H), lambda i: (i, 0)),
            vec, vec, vec, vec,
            pl.BlockSpec((1, 8, BE), lambda i: (i, 0, 0)),
            full((H, H)), full((H, H)), full((1, H)), full((8, H)),
            full((1, H)), full((H, H)), full((1, H)),
            full((H, H)), full((1, H)), full((1, H)),
        ],
        out_specs=[
            pl.BlockSpec((BE, H), lambda i: (i, 0)),
            vec, vec, vec,
        ],
        out_shape=[
            jax.ShapeDtypeStruct((E, H), jnp.float32),
            jax.ShapeDtypeStruct((E,), jnp.float32),
            jax.ShapeDtypeStruct((E,), jnp.float32),
            jax.ShapeDtypeStruct((E,), jnp.float32),
        ],
    )(ged, ges, rx, ry, rz, d2, eat,
      whd, whs, wd2, wea, b1, w2, b2, xw1, xb1, xw2r)


def _node_body(h_ref, p0_ref, p1_ref, cp_ref, q0_ref, q1_ref,
               w1a_ref, w1b_ref, b1_ref, w2_ref, b2_ref,
               hn_ref, cpn_ref):
    h = h_ref[...]
    aggm = p0_ref[...] + p1_ref[...]
    s4 = q0_ref[...] + q1_ref[...]
    deg = s4[:, 3:4]
    cpn_ref[...] = cp_ref[...] + s4 * (1.0 / (deg + 1.0))
    hu = jax.nn.silu(
        jnp.dot(h, w1a_ref[...], preferred_element_type=jnp.float32)
        + jnp.dot(aggm, w1b_ref[...], preferred_element_type=jnp.float32)
        + b1_ref[...])
    hn_ref[...] = h + jnp.dot(hu, w2_ref[...],
                              preferred_element_type=jnp.float32) + b2_ref[...]


def _node(h, p0, p1, cp4, q0, q1, w1a, w1b, b1, w2, b2):
    r = 400
    full = lambda shape: pl.BlockSpec(shape, lambda i: (0, 0))
    return pl.pallas_call(
        _node_body,
        grid=(N // r,),
        in_specs=[
            pl.BlockSpec((r, H), lambda i: (i, 0)),
            pl.BlockSpec((r, H), lambda i: (i, 0)),
            pl.BlockSpec((r, H), lambda i: (i, 0)),
            pl.BlockSpec((r, 4), lambda i: (i, 0)),
            pl.BlockSpec((r, 4), lambda i: (i, 0)),
            pl.BlockSpec((r, 4), lambda i: (i, 0)),
            full((H, H)), full((H, H)), full((1, H)),
            full((H, H)), full((1, H)),
        ],
        out_specs=[
            pl.BlockSpec((r, H), lambda i: (i, 0)),
            pl.BlockSpec((r, 4), lambda i: (i, 0)),
        ],
        out_shape=[
            jax.ShapeDtypeStruct((N, H), jnp.float32),
            jax.ShapeDtypeStruct((N, 4), jnp.float32),
        ],
    )(h, p0, p1, cp4, q0, q1, w1a, w1b, b1, w2, b2)


@jax.jit
def kernel(x, coords, edge_index, edge_attr, proj_w, proj_b,
           ew1, eb1, ew2, eb2, xw1, xb1, xw2, hw1, hb1, hw2, hb2):
    src = edge_index[0].astype(jnp.int32)
    dst = edge_index[1].astype(jnp.int32)
    eat = jnp.pad(edge_attr, ((0, 0), (0, 1))).reshape(NBLK, BE, 8)
    eat = eat.transpose(0, 2, 1)
    cp4 = jnp.pad(coords, ((0, 0), (0, 1)))
    cx, cy, cz = coords[:, 0], coords[:, 1], coords[:, 2]
    zm = jnp.zeros((N, H), jnp.float32)
    z4 = jnp.zeros((4 * N,), jnp.float32)

    h = _proj(x, proj_w, proj_b.reshape(1, H))
    for i in range(NLAYERS):
        ged, ges, rx, ry, rz, d2 = _gather(h, cx, cy, cz, src, dst)
        m, wx, wy, wz = _edge(ged, ges, rx, ry, rz, d2, eat,
                              ew1[i, :H], ew1[i, H:2 * H],
                              ew1[i, 2 * H:2 * H + 1],
                              jnp.pad(ew1[i, 2 * H + 1:], ((0, 1), (0, 0))),
                              eb1[i].reshape(1, H), ew2[i],
                              eb2[i].reshape(1, H),
                              xw1[i], xb1[i].reshape(1, H),
                              xw2[i].reshape(1, H))
        partm, part2 = _scatter(m, wx, wy, wz, dst, zm, z4)
        p2 = part2.reshape(NC, 4, N).transpose(0, 2, 1)
        h, cp4 = _node(h, partm[:N], partm[N:], cp4, p2[0], p2[1],
                       hw1[i, :H], hw1[i, H:], hb1[i].reshape(1, H),
                       hw2[i], hb2[i].reshape(1, H))
        cx, cy, cz = cp4[:, 0], cp4[:, 1], cp4[:, 2]
    return cp4[:, :3]
